# baseline (speedup 1.0000x reference)
import functools

import jax
import jax.numpy as jnp
from jax import lax
from jax.experimental import pallas as pl
from jax.experimental.pallas import tpu as pltpu
from jax.experimental.pallas import tpu_sc as plsc

_NUM_EXPERTS = 8
_BM = 256
_NC, _NS = 2, 16
_NW = _NC * _NS


def _routing(idx_flat, n_tok, top_k):
    S = n_tok * top_k
    P = S + _NUM_EXPERTS * _BM
    G = P // _BM
    e = idx_flat.reshape(-1).astype(jnp.int32)
    onehot = (e[:, None] == jnp.arange(_NUM_EXPERTS)[None, :]).astype(jnp.int32)
    cum = jnp.cumsum(onehot, axis=0)
    cnt = cum[-1]
    rank = jnp.sum(onehot * cum, axis=1) - 1
    padded = ((cnt + _BM - 1) // _BM) * _BM
    start = jnp.concatenate([jnp.zeros((1,), jnp.int32),
                             jnp.cumsum(padded)[:-1].astype(jnp.int32)])
    q = start[e] + rank
    src = jnp.zeros((P,), jnp.int32).at[q].set(
        jnp.arange(S, dtype=jnp.int32) // top_k)
    blk = jnp.sum(start[None, :] <= (jnp.arange(G, dtype=jnp.int32)[:, None] * _BM),
                  axis=1).astype(jnp.int32) - 1
    q2 = q.reshape(n_tok, top_k)
    return src, blk, q2[:, 0], q2[:, 1], P, G


def _make_gather(P, D, CH):
    rows_w = P // _NW
    n_ch = rows_w // CH
    mesh = plsc.VectorSubcoreMesh(core_axis_name="c", subcore_axis_name="s")

    @functools.partial(
        pl.kernel,
        out_type=jax.ShapeDtypeStruct((P, D), jnp.float32),
        mesh=mesh,
        scratch_types=[
            pltpu.VMEM((2, CH), jnp.int32),
            pltpu.VMEM((2, CH, D), jnp.float32),
            pltpu.SemaphoreType.DMA((2,)),
            pltpu.SemaphoreType.DMA((2,)),
        ],
    )
    def gather_k(x_hbm, src_hbm, xs_hbm, idx_v, rows_v, sem_g, sem_s):
        wid = lax.axis_index("s") * _NC + lax.axis_index("c")
        base = wid * rows_w

        def start_gather(c):
            sl = c % 2
            pltpu.sync_copy(src_hbm.at[pl.ds(base + c * CH, CH)], idx_v.at[sl])
            return pltpu.async_copy(x_hbm.at[idx_v.at[sl]], rows_v.at[sl],
                                    sem_g.at[sl])

        cps = {0: start_gather(0)}
        sts = {}
        for c in range(n_ch):
            if c + 1 < n_ch:
                if c - 1 in sts:
                    sts.pop(c - 1).wait()
                cps[c + 1] = start_gather(c + 1)
            cps.pop(c).wait()
            sts[c] = pltpu.async_copy(
                rows_v.at[c % 2], xs_hbm.at[pl.ds(base + c * CH, CH)],
                sem_s.at[c % 2])
        for c in sorted(sts):
            sts.pop(c).wait()

    return gather_k


def _mm_body(be_ref, xs_ref, w1_ref, b1_ref, w2_ref, b2_ref, out_ref):
    xb = xs_ref[...].astype(jnp.bfloat16)
    h = jnp.dot(xb, w1_ref[0], preferred_element_type=jnp.float32)
    h = h + b1_ref[0]
    h = 0.5 * h * (1.0 + jax.lax.erf(h * 0.7071067811865476))
    y = jnp.dot(h.astype(jnp.bfloat16), w2_ref[0],
                preferred_element_type=jnp.float32)
    out_ref[...] = y + b2_ref[0]


def _make_combine(P, D, n_tok):
    toks_w = n_tok // _NW
    CHT = 16
    n_ch = toks_w // CHT
    mesh = plsc.VectorSubcoreMesh(core_axis_name="c", subcore_axis_name="s")

    @functools.partial(
        pl.kernel,
        out_type=jax.ShapeDtypeStruct((n_tok, D), jnp.float32),
        mesh=mesh,
        scratch_types=[
            pltpu.VMEM((2, CHT), jnp.int32),
            pltpu.VMEM((2, CHT), jnp.int32),
            pltpu.VMEM((2, CHT, D), jnp.float32),
            pltpu.VMEM((2, CHT, D), jnp.float32),
            pltpu.SemaphoreType.DMA((2,)),
            pltpu.SemaphoreType.DMA((2,)),
            pltpu.SemaphoreType.DMA((2,)),
        ],
    )
    def combine_k(ys_hbm, qe_hbm, qo_hbm, out_hbm,
                  idx_a, idx_b, rows_a, rows_b, sem_a, sem_b, sem_s):
        wid = lax.axis_index("s") * _NC + lax.axis_index("c")
        base = wid * toks_w

        def start_gathers(c):
            sl = c % 2
            off = base + c * CHT
            pltpu.sync_copy(qe_hbm.at[pl.ds(off, CHT)], idx_a.at[sl])
            pltpu.sync_copy(qo_hbm.at[pl.ds(off, CHT)], idx_b.at[sl])
            return (pltpu.async_copy(ys_hbm.at[idx_a.at[sl]], rows_a.at[sl],
                                     sem_a.at[sl]),
                    pltpu.async_copy(ys_hbm.at[idx_b.at[sl]], rows_b.at[sl],
                                     sem_b.at[sl]))

        cps = {0: start_gathers(0)}
        sts = {}
        for c in range(n_ch):
            sl = c % 2
            if c + 1 < n_ch:
                if c - 1 in sts:
                    sts.pop(c - 1).wait()
                cps[c + 1] = start_gathers(c + 1)
            ca, cb = cps.pop(c)
            ca.wait()
            cb.wait()

            @pl.loop(0, CHT)
            def _row(i):
                @pl.loop(0, D // 16, unroll=8)
                def _vec(v):
                    s = pl.ds(v * 16, 16)
                    rows_a[sl, i, s] = rows_a[sl, i, s] + rows_b[sl, i, s]

            sts[c] = pltpu.async_copy(
                rows_a.at[sl], out_hbm.at[pl.ds(base + c * CHT, CHT)],
                sem_s.at[sl])
        for c in sorted(sts):
            sts.pop(c).wait()

    return combine_k


def kernel(x, expert_indices, W1, b1, W2, b2):
    batch, seq_len, d_model = x.shape
    n_tok = batch * seq_len
    d_ff = W1.shape[-1]
    top_k = expert_indices.shape[-1]

    x_flat = x.reshape(n_tok, d_model)
    idx_flat = expert_indices.reshape(n_tok, top_k).astype(jnp.int32)
    w1b = W1.astype(jnp.bfloat16)
    w2b = W2.astype(jnp.bfloat16)
    b1r = b1.reshape(_NUM_EXPERTS, 1, d_ff)
    b2r = b2.reshape(_NUM_EXPERTS, 1, d_model)

    src, blk, qe, qo, P, G = _routing(idx_flat, n_tok, top_k)

    NSPLIT = 4
    Pq = P // NSPLIT
    Gq = G // NSPLIT
    gather_q = _make_gather(Pq, d_model, CH=48)
    xs_parts = [
        gather_q(x_flat, lax.slice(src, (i * Pq,), ((i + 1) * Pq,)))
        for i in range(NSPLIT)
    ]

    ys = None
    for i in range(NSPLIT):
        off = i * Gq
        blk_i = lax.slice(blk, (off,), (off + Gq,))
        in_specs = [
            pl.BlockSpec((_BM, d_model), lambda g, be: (g, 0)),
            pl.BlockSpec((1, d_model, d_ff), lambda g, be: (be[g], 0, 0)),
            pl.BlockSpec((1, 1, d_ff), lambda g, be: (be[g], 0, 0)),
            pl.BlockSpec((1, d_ff, d_model), lambda g, be: (be[g], 0, 0)),
            pl.BlockSpec((1, 1, d_model), lambda g, be: (be[g], 0, 0)),
        ]
        args = [blk_i, xs_parts[i], w1b, b1r, w2b, b2r]
        kwargs = {}
        body = _mm_body
        if ys is not None:
            in_specs.append(pl.BlockSpec(memory_space=pl.ANY))
            args.append(ys)
            kwargs["input_output_aliases"] = {6: 0}

            def body(be_ref, xs_ref, w1_ref, b1_ref, w2_ref, b2_ref,
                     ys_ref, out_ref):
                del ys_ref
                _mm_body(be_ref, xs_ref, w1_ref, b1_ref, w2_ref, b2_ref,
                         out_ref)

        grid_spec = pltpu.PrefetchScalarGridSpec(
            num_scalar_prefetch=1,
            grid=(Gq,),
            in_specs=in_specs,
            out_specs=pl.BlockSpec((_BM, d_model),
                                   lambda g, be, o=off // Gq: (g + o * Gq, 0)),
        )
        ys = pl.pallas_call(
            body,
            grid_spec=grid_spec,
            out_shape=jax.ShapeDtypeStruct((P, d_model), jnp.float32),
            compiler_params=pltpu.CompilerParams(
                dimension_semantics=("arbitrary",),
            ),
            **kwargs,
        )(*args)

    out = _make_combine(P, d_model, n_tok)(ys, qe, qo)
    return out.reshape(batch, seq_len, d_model)


# speedup vs baseline: 1.0352x; 1.0352x over previous
import functools

import jax
import jax.numpy as jnp
from jax import lax
from jax.experimental import pallas as pl
from jax.experimental.pallas import tpu as pltpu
from jax.experimental.pallas import tpu_sc as plsc

_NUM_EXPERTS = 8
_BM = 256
_NC, _NS = 2, 16
_NW = _NC * _NS


def _routing(idx_flat, n_tok, top_k):
    S = n_tok * top_k
    P = S + _NUM_EXPERTS * _BM
    G = P // _BM
    e = idx_flat.reshape(-1).astype(jnp.int32)
    onehot = (e[:, None] == jnp.arange(_NUM_EXPERTS)[None, :]).astype(jnp.int32)
    cum = jnp.cumsum(onehot, axis=0)
    cnt = cum[-1]
    rank = jnp.sum(onehot * cum, axis=1) - 1
    padded = ((cnt + _BM - 1) // _BM) * _BM
    start = jnp.concatenate([jnp.zeros((1,), jnp.int32),
                             jnp.cumsum(padded)[:-1].astype(jnp.int32)])
    q = start[e] + rank
    src = jnp.zeros((P,), jnp.int32).at[q].set(
        jnp.arange(S, dtype=jnp.int32) // top_k)
    blk = jnp.sum(start[None, :] <= (jnp.arange(G, dtype=jnp.int32)[:, None] * _BM),
                  axis=1).astype(jnp.int32) - 1
    q2 = q.reshape(n_tok, top_k)
    return src, blk, q2[:, 0], q2[:, 1], P, G


def _make_gather(P, D):
    rows_w = P // _NW
    CH = 32
    NBUF = 3
    n_ch = rows_w // CH
    mesh = plsc.VectorSubcoreMesh(core_axis_name="c", subcore_axis_name="s")

    @functools.partial(
        pl.kernel,
        out_type=jax.ShapeDtypeStruct((P, D), jnp.float32),
        mesh=mesh,
        scratch_types=[
            pltpu.VMEM((rows_w,), jnp.int32),
            pltpu.VMEM((NBUF, CH, D), jnp.float32),
            pltpu.SemaphoreType.DMA((NBUF,)),
            pltpu.SemaphoreType.DMA((NBUF,)),
        ],
    )
    def gather_k(x_hbm, src_hbm, xs_hbm, idx_all, rows_v, sem_g, sem_s):
        wid = lax.axis_index("s") * _NC + lax.axis_index("c")
        base = wid * rows_w
        pltpu.sync_copy(src_hbm.at[pl.ds(base, rows_w)], idx_all)

        def start_gather(c):
            return pltpu.async_copy(
                x_hbm.at[idx_all.at[pl.ds(c * CH, CH)]],
                rows_v.at[c % NBUF], sem_g.at[c % NBUF])

        cps = {c: start_gather(c) for c in range(min(NBUF - 1, n_ch))}
        sts = {}
        for c in range(n_ch):
            nxt = c + NBUF - 1
            if nxt < n_ch:
                prev = nxt - NBUF
                if prev in sts:
                    sts.pop(prev).wait()
                cps[nxt] = start_gather(nxt)
            cps.pop(c).wait()
            sts[c] = pltpu.async_copy(
                rows_v.at[c % NBUF], xs_hbm.at[pl.ds(base + c * CH, CH)],
                sem_s.at[c % NBUF])
        for c in sorted(sts):
            sts.pop(c).wait()

    return gather_k


def _mm_body(be_ref, xs_ref, w1_ref, b1_ref, w2_ref, b2_ref, out_ref):
    xb = xs_ref[...].astype(jnp.bfloat16)
    h = jnp.dot(xb, w1_ref[0], preferred_element_type=jnp.float32)
    h = h + b1_ref[0]
    h = 0.5 * h * (1.0 + jax.lax.erf(h * 0.7071067811865476))
    y = jnp.dot(h.astype(jnp.bfloat16), w2_ref[0],
                preferred_element_type=jnp.float32)
    out_ref[...] = y + b2_ref[0]


def _make_combine(P, D, n_tok):
    toks_w = n_tok // _NW
    CHT = 16
    n_ch = toks_w // CHT
    mesh = plsc.VectorSubcoreMesh(core_axis_name="c", subcore_axis_name="s")

    @functools.partial(
        pl.kernel,
        out_type=jax.ShapeDtypeStruct((n_tok, D), jnp.float32),
        mesh=mesh,
        scratch_types=[
            pltpu.VMEM((toks_w,), jnp.int32),
            pltpu.VMEM((toks_w,), jnp.int32),
            pltpu.VMEM((2, CHT, D), jnp.float32),
            pltpu.VMEM((2, CHT, D), jnp.float32),
            pltpu.SemaphoreType.DMA((2,)),
            pltpu.SemaphoreType.DMA((2,)),
            pltpu.SemaphoreType.DMA((2,)),
        ],
    )
    def combine_k(ys_hbm, qe_hbm, qo_hbm, out_hbm,
                  idx_a, idx_b, rows_a, rows_b, sem_a, sem_b, sem_s):
        wid = lax.axis_index("s") * _NC + lax.axis_index("c")
        base = wid * toks_w
        pltpu.sync_copy(qe_hbm.at[pl.ds(base, toks_w)], idx_a)
        pltpu.sync_copy(qo_hbm.at[pl.ds(base, toks_w)], idx_b)

        def start_gathers(c):
            sl = c % 2
            s_idx = pl.ds(c * CHT, CHT)
            return (pltpu.async_copy(ys_hbm.at[idx_a.at[s_idx]], rows_a.at[sl],
                                     sem_a.at[sl]),
                    pltpu.async_copy(ys_hbm.at[idx_b.at[s_idx]], rows_b.at[sl],
                                     sem_b.at[sl]))

        cps = {0: start_gathers(0)}
        sts = {}
        for c in range(n_ch):
            sl = c % 2
            if c + 1 < n_ch:
                if c - 1 in sts:
                    sts.pop(c - 1).wait()
                cps[c + 1] = start_gathers(c + 1)
            ca, cb = cps.pop(c)
            ca.wait()
            cb.wait()

            @pl.loop(0, CHT)
            def _row(i):
                @pl.loop(0, D // 16, unroll=8)
                def _vec(v):
                    s = pl.ds(v * 16, 16)
                    rows_a[sl, i, s] = rows_a[sl, i, s] + rows_b[sl, i, s]

            sts[c] = pltpu.async_copy(
                rows_a.at[sl], out_hbm.at[pl.ds(base + c * CHT, CHT)],
                sem_s.at[sl])
        for c in sorted(sts):
            sts.pop(c).wait()

    return combine_k


def kernel(x, expert_indices, W1, b1, W2, b2):
    batch, seq_len, d_model = x.shape
    n_tok = batch * seq_len
    d_ff = W1.shape[-1]
    top_k = expert_indices.shape[-1]

    x_flat = x.reshape(n_tok, d_model)
    idx_flat = expert_indices.reshape(n_tok, top_k).astype(jnp.int32)
    w1b = W1.astype(jnp.bfloat16)
    w2b = W2.astype(jnp.bfloat16)
    b1r = b1.reshape(_NUM_EXPERTS, 1, d_ff)
    b2r = b2.reshape(_NUM_EXPERTS, 1, d_model)

    src, blk, qe, qo, P, G = _routing(idx_flat, n_tok, top_k)

    xs = _make_gather(P, d_model)(x_flat, src)

    grid_spec = pltpu.PrefetchScalarGridSpec(
        num_scalar_prefetch=1,
        grid=(G,),
        in_specs=[
            pl.BlockSpec((_BM, d_model), lambda g, be: (g, 0)),
            pl.BlockSpec((1, d_model, d_ff), lambda g, be: (be[g], 0, 0)),
            pl.BlockSpec((1, 1, d_ff), lambda g, be: (be[g], 0, 0)),
            pl.BlockSpec((1, d_ff, d_model), lambda g, be: (be[g], 0, 0)),
            pl.BlockSpec((1, 1, d_model), lambda g, be: (be[g], 0, 0)),
        ],
        out_specs=pl.BlockSpec((_BM, d_model), lambda g, be: (g, 0)),
    )
    ys = pl.pallas_call(
        _mm_body,
        grid_spec=grid_spec,
        out_shape=jax.ShapeDtypeStruct((P, d_model), jnp.float32),
        compiler_params=pltpu.CompilerParams(
            dimension_semantics=("arbitrary",),
        ),
    )(blk, xs, w1b, b1r, w2b, b2r)

    out = _make_combine(P, d_model, n_tok)(ys, qe, qo)
    return out.reshape(batch, seq_len, d_model)


# speedup vs baseline: 1.0425x; 1.0071x over previous
import functools

import jax
import jax.numpy as jnp
from jax import lax
from jax.experimental import pallas as pl
from jax.experimental.pallas import tpu as pltpu
from jax.experimental.pallas import tpu_sc as plsc

_NUM_EXPERTS = 8
_BM = 256
_NC, _NS = 2, 16
_NW = _NC * _NS


def _routing(idx_flat, n_tok, top_k):
    S = n_tok * top_k
    P = S + _NUM_EXPERTS * _BM
    G = P // _BM
    e = idx_flat.reshape(-1).astype(jnp.int32)
    onehot = (e[:, None] == jnp.arange(_NUM_EXPERTS)[None, :]).astype(jnp.int32)
    cum = jnp.cumsum(onehot, axis=0)
    cnt = cum[-1]
    rank = jnp.sum(onehot * cum, axis=1) - 1
    padded = ((cnt + _BM - 1) // _BM) * _BM
    start = jnp.concatenate([jnp.zeros((1,), jnp.int32),
                             jnp.cumsum(padded)[:-1].astype(jnp.int32)])
    q = start[e] + rank
    src = jnp.zeros((P,), jnp.int32).at[q].set(
        jnp.arange(S, dtype=jnp.int32) // top_k)
    gstart = jnp.arange(G, dtype=jnp.int32) * _BM
    blk = jnp.sum(start[None, :] <= gstart[:, None],
                  axis=1).astype(jnp.int32) - 1
    work = (gstart < (start + cnt)[blk]).astype(jnp.int32)
    q2 = q.reshape(n_tok, top_k)
    return src, blk, work, q2[:, 0], q2[:, 1], P, G


def _make_gather(P, D):
    rows_w = P // _NW
    CH = 32
    NBUF = 3
    n_ch = rows_w // CH
    mesh = plsc.VectorSubcoreMesh(core_axis_name="c", subcore_axis_name="s")

    @functools.partial(
        pl.kernel,
        out_type=jax.ShapeDtypeStruct((P, D), jnp.float32),
        mesh=mesh,
        scratch_types=[
            pltpu.VMEM((rows_w,), jnp.int32),
            pltpu.VMEM((NBUF, CH, D), jnp.float32),
            pltpu.SemaphoreType.DMA((NBUF,)),
            pltpu.SemaphoreType.DMA((NBUF,)),
        ],
    )
    def gather_k(x_hbm, src_hbm, xs_hbm, idx_all, rows_v, sem_g, sem_s):
        wid = lax.axis_index("s") * _NC + lax.axis_index("c")
        base = wid * rows_w
        pltpu.sync_copy(src_hbm.at[pl.ds(base, rows_w)], idx_all)

        def start_gather(c):
            return pltpu.async_copy(
                x_hbm.at[idx_all.at[pl.ds(c * CH, CH)]],
                rows_v.at[c % NBUF], sem_g.at[c % NBUF])

        cps = {c: start_gather(c) for c in range(min(NBUF - 1, n_ch))}
        sts = {}
        for c in range(n_ch):
            nxt = c + NBUF - 1
            if nxt < n_ch:
                prev = nxt - NBUF
                if prev in sts:
                    sts.pop(prev).wait()
                cps[nxt] = start_gather(nxt)
            cps.pop(c).wait()
            sts[c] = pltpu.async_copy(
                rows_v.at[c % NBUF], xs_hbm.at[pl.ds(base + c * CH, CH)],
                sem_s.at[c % NBUF])
        for c in sorted(sts):
            sts.pop(c).wait()

    return gather_k


def _mm_body(be_ref, work_ref, xs_ref, w1_ref, b1_ref, w2_ref, b2_ref,
             out_ref):
    @pl.when(work_ref[pl.program_id(0)] == 1)
    def _():
        xb = xs_ref[...].astype(jnp.bfloat16)
        h = jnp.dot(xb, w1_ref[0], preferred_element_type=jnp.float32)
        h = h + b1_ref[0]
        h = 0.5 * h * (1.0 + jax.lax.erf(h * 0.7071067811865476))
        y = jnp.dot(h.astype(jnp.bfloat16), w2_ref[0],
                    preferred_element_type=jnp.float32)
        out_ref[...] = y + b2_ref[0]


def _make_combine(P, D, n_tok):
    toks_w = n_tok // _NW
    CHT = 16
    n_ch = toks_w // CHT
    mesh = plsc.VectorSubcoreMesh(core_axis_name="c", subcore_axis_name="s")

    @functools.partial(
        pl.kernel,
        out_type=jax.ShapeDtypeStruct((n_tok, D), jnp.float32),
        mesh=mesh,
        scratch_types=[
            pltpu.VMEM((toks_w,), jnp.int32),
            pltpu.VMEM((toks_w,), jnp.int32),
            pltpu.VMEM((2, CHT, D), jnp.float32),
            pltpu.VMEM((2, CHT, D), jnp.float32),
            pltpu.SemaphoreType.DMA((2,)),
            pltpu.SemaphoreType.DMA((2,)),
            pltpu.SemaphoreType.DMA((2,)),
        ],
    )
    def combine_k(ys_hbm, qe_hbm, qo_hbm, out_hbm,
                  idx_a, idx_b, rows_a, rows_b, sem_a, sem_b, sem_s):
        wid = lax.axis_index("s") * _NC + lax.axis_index("c")
        base = wid * toks_w
        pltpu.sync_copy(qe_hbm.at[pl.ds(base, toks_w)], idx_a)
        pltpu.sync_copy(qo_hbm.at[pl.ds(base, toks_w)], idx_b)

        def start_gathers(c):
            sl = c % 2
            s_idx = pl.ds(c * CHT, CHT)
            return (pltpu.async_copy(ys_hbm.at[idx_a.at[s_idx]], rows_a.at[sl],
                                     sem_a.at[sl]),
                    pltpu.async_copy(ys_hbm.at[idx_b.at[s_idx]], rows_b.at[sl],
                                     sem_b.at[sl]))

        cps = {0: start_gathers(0)}
        sts = {}
        for c in range(n_ch):
            sl = c % 2
            if c + 1 < n_ch:
                if c - 1 in sts:
                    sts.pop(c - 1).wait()
                cps[c + 1] = start_gathers(c + 1)
            ca, cb = cps.pop(c)
            ca.wait()
            cb.wait()

            @pl.loop(0, CHT)
            def _row(i):
                @pl.loop(0, D // 16, unroll=8)
                def _vec(v):
                    s = pl.ds(v * 16, 16)
                    rows_a[sl, i, s] = rows_a[sl, i, s] + rows_b[sl, i, s]

            sts[c] = pltpu.async_copy(
                rows_a.at[sl], out_hbm.at[pl.ds(base + c * CHT, CHT)],
                sem_s.at[sl])
        for c in sorted(sts):
            sts.pop(c).wait()

    return combine_k


def kernel(x, expert_indices, W1, b1, W2, b2):
    batch, seq_len, d_model = x.shape
    n_tok = batch * seq_len
    d_ff = W1.shape[-1]
    top_k = expert_indices.shape[-1]

    x_flat = x.reshape(n_tok, d_model)
    idx_flat = expert_indices.reshape(n_tok, top_k).astype(jnp.int32)
    w1b = W1.astype(jnp.bfloat16)
    w2b = W2.astype(jnp.bfloat16)
    b1r = b1.reshape(_NUM_EXPERTS, 1, d_ff)
    b2r = b2.reshape(_NUM_EXPERTS, 1, d_model)

    src, blk, work, qe, qo, P, G = _routing(idx_flat, n_tok, top_k)

    xs = _make_gather(P, d_model)(x_flat, src)

    grid_spec = pltpu.PrefetchScalarGridSpec(
        num_scalar_prefetch=2,
        grid=(G,),
        in_specs=[
            pl.BlockSpec((_BM, d_model), lambda g, be, wk: (g, 0)),
            pl.BlockSpec((1, d_model, d_ff), lambda g, be, wk: (be[g], 0, 0)),
            pl.BlockSpec((1, 1, d_ff), lambda g, be, wk: (be[g], 0, 0)),
            pl.BlockSpec((1, d_ff, d_model), lambda g, be, wk: (be[g], 0, 0)),
            pl.BlockSpec((1, 1, d_model), lambda g, be, wk: (be[g], 0, 0)),
        ],
        out_specs=pl.BlockSpec((_BM, d_model), lambda g, be, wk: (g, 0)),
    )
    ys = pl.pallas_call(
        _mm_body,
        grid_spec=grid_spec,
        out_shape=jax.ShapeDtypeStruct((P, d_model), jnp.float32),
        compiler_params=pltpu.CompilerParams(
            dimension_semantics=("arbitrary",),
        ),
    )(blk, work, xs, w1b, b1r, w2b, b2r)

    out = _make_combine(P, d_model, n_tok)(ys, qe, qo)
    return out.reshape(batch, seq_len, d_model)
